# Initial kernel scaffold; baseline (speedup 1.0000x reference)
import jax
import jax.numpy as jnp
from jax.experimental import pallas as pl


def kernel(input_ids, token_type_ids, word_emb, pos_emb, type_emb, gamma, beta):
    raise NotImplementedError("write your pallas kernel here")


# speedup vs baseline: 2.7180x; 2.7180x over previous
import functools

import jax
import jax.numpy as jnp
from jax import lax
from jax.experimental import pallas as pl
from jax.experimental.pallas import tpu as pltpu
from jax.experimental.pallas import tpu_sc as plsc

HIDDEN = 768
EPS = 1e-5
NUM_WORKERS = 32


def _sc_gather(table, idx):
    _, D = table.shape
    B = idx.shape[0]
    b_per_w = B // NUM_WORKERS
    C = 128
    n_chunks = b_per_w // C
    mesh = plsc.VectorSubcoreMesh(core_axis_name="c", subcore_axis_name="s")

    @functools.partial(
        pl.kernel, mesh=mesh,
        out_type=jax.ShapeDtypeStruct((B, D), jnp.float32),
        scratch_types=[
            pltpu.VMEM((C,), jnp.int32),
            pltpu.VMEM((C, D), jnp.float32),
            pltpu.SemaphoreType.DMA,
        ],
    )
    def k(table_hbm, idx_hbm, out_hbm, idx_v, rows_v, sem):
        wid = lax.axis_index("s") * 2 + lax.axis_index("c")
        base = wid * b_per_w

        def body(i, carry):
            off = base + i * C
            pltpu.sync_copy(idx_hbm.at[pl.ds(off, C)], idx_v)
            pltpu.async_copy(table_hbm.at[idx_v], rows_v, sem).wait()
            pltpu.sync_copy(rows_v, out_hbm.at[pl.ds(off, C)])
            return carry

        lax.fori_loop(0, n_chunks, body, 0)

    return k(table, idx)


def _tc_layernorm(x, pos_emb, tt3, type_emb, gamma2, beta2):
    BATCH, SEQ, _ = x.shape

    def body(x_ref, pos_ref, tt_ref, type_ref, g_ref, b_ref, o_ref):
        xb = x_ref[0]
        pos = pos_ref[...]
        ttc = tt_ref[0]
        t0 = type_ref[0]
        t1 = type_ref[1]
        e = xb + pos + (t0[None, :] * (1.0 - ttc) + t1[None, :] * ttc)
        mean = jnp.mean(e, axis=-1, keepdims=True)
        c = e - mean
        var = jnp.mean(c * c, axis=-1, keepdims=True)
        o_ref[0] = c * lax.rsqrt(var + EPS) * g_ref[0][None, :] + b_ref[0][None, :]

    return pl.pallas_call(
        body,
        grid=(BATCH,),
        in_specs=[
            pl.BlockSpec((1, SEQ, HIDDEN), lambda b: (b, 0, 0)),
            pl.BlockSpec((SEQ, HIDDEN), lambda b: (0, 0)),
            pl.BlockSpec((1, SEQ, 1), lambda b: (b, 0, 0)),
            pl.BlockSpec((2, HIDDEN), lambda b: (0, 0)),
            pl.BlockSpec((1, HIDDEN), lambda b: (0, 0)),
            pl.BlockSpec((1, HIDDEN), lambda b: (0, 0)),
        ],
        out_specs=pl.BlockSpec((1, SEQ, HIDDEN), lambda b: (b, 0, 0)),
        out_shape=jax.ShapeDtypeStruct((BATCH, SEQ, HIDDEN), jnp.float32),
    )(x, pos_emb, tt3, type_emb, gamma2, beta2)


def kernel(input_ids, token_type_ids, word_emb, pos_emb, type_emb, gamma, beta):
    B, S = input_ids.shape
    ids = input_ids.reshape(-1).astype(jnp.int32)
    gathered = _sc_gather(word_emb, ids)
    x = gathered.reshape(B, S, HIDDEN)
    tt3 = token_type_ids.reshape(B, S, 1).astype(jnp.float32)
    return _tc_layernorm(
        x, pos_emb, tt3, type_emb,
        gamma.reshape(1, HIDDEN), beta.reshape(1, HIDDEN),
    )


# speedup vs baseline: 3.1069x; 1.1431x over previous
import functools

import jax
import jax.numpy as jnp
from jax import lax
from jax.experimental import pallas as pl
from jax.experimental.pallas import tpu as pltpu
from jax.experimental.pallas import tpu_sc as plsc

HIDDEN = 768
EPS = 1e-5
NUM_WORKERS = 32


def _sc_gather(table, idx):
    _, D = table.shape
    B = idx.shape[0]
    b_per_w = B // NUM_WORKERS
    C = 128
    n_chunks = b_per_w // C
    mesh = plsc.VectorSubcoreMesh(core_axis_name="c", subcore_axis_name="s")

    @functools.partial(
        pl.kernel, mesh=mesh,
        out_type=jax.ShapeDtypeStruct((B, D), jnp.float32),
        scratch_types=[
            pltpu.VMEM((C,), jnp.int32),
            pltpu.VMEM((C, D), jnp.float32),
            pltpu.SemaphoreType.DMA,
        ],
    )
    def k(table_hbm, idx_hbm, out_hbm, idx_v, rows_v, sem):
        wid = lax.axis_index("s") * 2 + lax.axis_index("c")
        base = wid * b_per_w

        def body(i, carry):
            off = base + i * C
            pltpu.sync_copy(idx_hbm.at[pl.ds(off, C)], idx_v)
            pltpu.async_copy(table_hbm.at[idx_v], rows_v, sem).wait()
            pltpu.sync_copy(rows_v, out_hbm.at[pl.ds(off, C)])
            return carry

        lax.fori_loop(0, n_chunks, body, 0)

    return k(table, idx)


def _tc_layernorm(x, pos_emb, tt3, type_emb, gamma2, beta2):
    BATCH, SEQ, _ = x.shape
    BB = 4

    def body(x_ref, pos_ref, tt_ref, type_ref, g_ref, b_ref, o_ref):
        pos = pos_ref[...]
        t0 = type_ref[0]
        t1 = type_ref[1]
        g = g_ref[0]
        bb = b_ref[0]
        for i in range(BB):
            xb = x_ref[i]
            ttc = tt_ref[i]
            e = xb + pos + (t0[None, :] * (1.0 - ttc) + t1[None, :] * ttc)
            mean = jnp.mean(e, axis=-1, keepdims=True)
            c = e - mean
            var = jnp.mean(c * c, axis=-1, keepdims=True)
            o_ref[i] = c * lax.rsqrt(var + EPS) * g[None, :] + bb[None, :]

    return pl.pallas_call(
        body,
        grid=(BATCH // BB,),
        in_specs=[
            pl.BlockSpec((BB, SEQ, HIDDEN), lambda b: (b, 0, 0)),
            pl.BlockSpec((SEQ, HIDDEN), lambda b: (0, 0)),
            pl.BlockSpec((BB, SEQ, 1), lambda b: (b, 0, 0)),
            pl.BlockSpec((2, HIDDEN), lambda b: (0, 0)),
            pl.BlockSpec((1, HIDDEN), lambda b: (0, 0)),
            pl.BlockSpec((1, HIDDEN), lambda b: (0, 0)),
        ],
        out_specs=pl.BlockSpec((BB, SEQ, HIDDEN), lambda b: (b, 0, 0)),
        out_shape=jax.ShapeDtypeStruct((BATCH, SEQ, HIDDEN), jnp.float32),
    )(x, pos_emb, tt3, type_emb, gamma2, beta2)


def kernel(input_ids, token_type_ids, word_emb, pos_emb, type_emb, gamma, beta):
    B, S = input_ids.shape
    ids = input_ids.reshape(-1).astype(jnp.int32)
    gathered = _sc_gather(word_emb, ids)
    x = gathered.reshape(B, S, HIDDEN)
    tt3 = token_type_ids.reshape(B, S, 1).astype(jnp.float32)
    return _tc_layernorm(
        x, pos_emb, tt3, type_emb,
        gamma.reshape(1, HIDDEN), beta.reshape(1, HIDDEN),
    )
